# baseline (speedup 1.0000x reference)
import functools

import jax
import jax.numpy as jnp
from jax import lax
from jax.experimental import pallas as pl
from jax.experimental.pallas import tpu as pltpu
from jax.experimental.pallas import tpu_sc as plsc

_B = 16384
_D = 64
_V = 100000

_info = plsc.get_sparse_core_info()
_NC = _info.num_cores
_NS = _info.num_subcores
_NW = _NC * _NS
_DPW = _D // (_NW // 2)
_VMAIN = (_V // 128) * 128
_TAIL = _V - _VMAIN
_SLAB_LO = (0, 33408, 66816)
_SLAB_SZ = (33408, 33408, _V - 66816)
_SLAB_DMA = (33408, 33408, _VMAIN - 66816)
_SLAB_MAX = max(_SLAB_SZ)


def _slab_pass(buf, idx_v, out_v, lo, size, first):
    @plsc.parallel_loop(0, _B // 16, unroll=8)
    def _(g):
        iv = idx_v[pl.ds(g * 16, 16)]
        rel = iv - lo
        mask = plsc.bitcast(rel, jnp.uint32) < jnp.uint32(size)
        vals = plsc.load_gather(buf, [rel], mask=mask)
        if first:
            out_v[pl.ds(g * 16, 16)] = vals
        else:
            pos = lax.iota(jnp.int32, 16) + g * 16
            plsc.store_scatter(out_v, [pos], vals, mask=mask)


def _do_table(tbl, tail_f, out2d, d_base, idx_v, bufs, out_v, sems):
    tasks = [(t, s) for t in range(_DPW) for s in range(3)]
    n = len(tasks)

    def issue(k):
        t, s = tasks[k]
        buf = bufs[k % 2]
        sem = sems[k % 2]
        copies = [pltpu.async_copy(
            tbl.at[d_base + t, pl.ds(_SLAB_LO[s], _SLAB_DMA[s])],
            buf.at[pl.ds(0, _SLAB_DMA[s])], sem)]
        if s == 2:
            copies.append(pltpu.async_copy(
                tail_f.at[pl.ds((d_base + t) * _TAIL, _TAIL)],
                buf.at[pl.ds(_SLAB_DMA[s], _TAIL)], sem))
        return copies

    cur = issue(0)
    for k, (t, s) in enumerate(tasks):
        nxt = issue(k + 1) if k + 1 < n else None
        for c in cur:
            c.wait()
        _slab_pass(bufs[k % 2], idx_v, out_v, _SLAB_LO[s], _SLAB_SZ[s],
                   first=(s == 0))
        if s == 2:
            pltpu.sync_copy(out_v, out2d.at[d_base + t])
        cur = nxt


def _gather_body(ugT_t, igT_t, ub_t, ib_t, tailu_f, taili_f, users, items,
                 ugT_o, igT_o, ub_o, ib_o,
                 buf_a, buf_b, idx_v, out_v, sem_a, sem_b, sem_x):
    wid = lax.axis_index("s") * _NC + lax.axis_index("c")
    is_user = wid < (_NW // 2)
    local = lax.rem(wid, _NW // 2)
    d_base = local * _DPW

    @pl.when(is_user)
    def _():
        pltpu.sync_copy(users, idx_v)
        _do_table(ugT_t, tailu_f, ugT_o, d_base, idx_v, (buf_a, buf_b),
                  out_v, (sem_a, sem_b))

    @pl.when(jnp.logical_not(is_user))
    def _():
        pltpu.sync_copy(items, idx_v)
        _do_table(igT_t, taili_f, igT_o, d_base, idx_v, (buf_a, buf_b),
                  out_v, (sem_a, sem_b))

    bpt = _B // (_NW // 2)
    base = local * bpt

    def beta_gather(beta_t, beta_o):
        copies = [
            pltpu.async_copy(beta_t.at[idx_v.at[pl.ds(base + c * 128, 128)]],
                             out_v.at[pl.ds(c * 128, 128)], sem_x)
            for c in range(bpt // 128)
        ]
        for c in copies:
            c.wait()
        pltpu.sync_copy(out_v.at[pl.ds(0, bpt)], beta_o.at[pl.ds(base, bpt)])

    @pl.when(is_user)
    def _():
        beta_gather(ub_t, ub_o)

    @pl.when(jnp.logical_not(is_user))
    def _():
        beta_gather(ib_t, ib_o)


_sc_gather = functools.partial(
    pl.kernel,
    mesh=plsc.VectorSubcoreMesh(core_axis_name="c", subcore_axis_name="s"),
    out_type=[
        jax.ShapeDtypeStruct((_D, _B), jnp.float32),
        jax.ShapeDtypeStruct((_D, _B), jnp.float32),
        jax.ShapeDtypeStruct((_B,), jnp.float32),
        jax.ShapeDtypeStruct((_B,), jnp.float32),
    ],
    scratch_types=[
        pltpu.VMEM((_SLAB_MAX,), jnp.float32),
        pltpu.VMEM((_SLAB_MAX,), jnp.float32),
        pltpu.VMEM((_B,), jnp.int32),
        pltpu.VMEM((_B,), jnp.float32),
        pltpu.SemaphoreType.DMA,
        pltpu.SemaphoreType.DMA,
        pltpu.SemaphoreType.DMA,
    ],
    compiler_params=pltpu.CompilerParams(needs_layout_passes=False),
)(_gather_body)


def _math_body(ug_ref, ig_ref, ub_ref, ib_ref, out_ref):
    ug = ug_ref[...]
    ig = ig_ref[...]
    cu = jnp.maximum(jnp.sqrt(jnp.sum(ug * ug, axis=1, keepdims=True)), 1e-12)
    ci = jnp.maximum(jnp.sqrt(jnp.sum(ig * ig, axis=1, keepdims=True)), 1e-12)
    w = 1.0 / (cu * ci)
    wu = 1.0 / (cu * cu)
    wi = 1.0 / (ci * ci)
    num = jnp.sum(ug * ig * w, axis=0)
    rnu = jnp.sqrt(jnp.sum(ug * ug * wu, axis=0))
    rni = jnp.sqrt(jnp.sum(ig * ig * wi, axis=0))
    den = jnp.maximum(rnu, 1e-8) * jnp.maximum(rni, 1e-8)
    ub = ub_ref[...]
    ib = ib_ref[...]
    nbu = jnp.maximum(jnp.sqrt(jnp.sum(ub * ub)), 1e-12)
    nbi = jnp.maximum(jnp.sqrt(jnp.sum(ib * ib)), 1e-12)
    out_ref[...] = ib / nbi + ub / nbu + num / den


_tc_math = pl.pallas_call(
    _math_body,
    out_shape=jax.ShapeDtypeStruct((_B,), jnp.float32),
)


def kernel(users, items, user_gama, item_gama, user_beta, item_beta):
    users = users.astype(jnp.int32)
    items = items.astype(jnp.int32)
    ugT_t = user_gama.T
    igT_t = item_gama.T
    ub_t = user_beta.reshape(-1)
    ib_t = item_beta.reshape(-1)
    tailu_f = user_gama[_VMAIN:, :].T.reshape(-1)
    taili_f = item_gama[_VMAIN:, :].T.reshape(-1)
    ugT, igT, ub, ib = _sc_gather(ugT_t, igT_t, ub_t, ib_t,
                                  tailu_f, taili_f, users, items)
    return _tc_math(ugT, igT, ub, ib)


# speedup vs baseline: 1.0298x; 1.0298x over previous
import functools

import jax
import jax.numpy as jnp
from jax import lax
from jax.experimental import pallas as pl
from jax.experimental.pallas import tpu as pltpu
from jax.experimental.pallas import tpu_sc as plsc

_B = 16384
_D = 64
_V = 100000

_info = plsc.get_sparse_core_info()
_NC = _info.num_cores
_NS = _info.num_subcores
_NW = _NC * _NS
_DPW = _D // (_NW // 2)
_SEG = 8192
_NSEG = _B // _SEG


def _gather_rows(row_v, idx_v, out_v, write_seg):
    for seg in range(_NSEG):
        @plsc.parallel_loop(0, _SEG // 16, unroll=8)
        def _(g):
            iv = idx_v[pl.ds(seg * _SEG + g * 16, 16)]
            out_v[pl.ds(g * 16, 16)] = plsc.load_gather(row_v, [iv])
        write_seg(seg)


def _gather_body(ugT_t, igT_t, ub_t, ib_t, users, items,
                 ugT_o, igT_o, ub_o, ib_o,
                 row_v, idx_v, out_v, sem):
    wid = lax.axis_index("s") * _NC + lax.axis_index("c")
    is_user = wid < (_NW // 2)
    local = lax.rem(wid, _NW // 2)
    d_base = local * _DPW

    @pl.when(is_user)
    def _():
        pltpu.sync_copy(users, idx_v)

    @pl.when(jnp.logical_not(is_user))
    def _():
        pltpu.sync_copy(items, idx_v)

    for t in range(_DPW):
        d = d_base + t

        @pl.when(is_user)
        def _():
            pltpu.sync_copy(ugT_t.at[d], row_v)

            def write_seg(seg):
                pltpu.sync_copy(out_v, ugT_o.at[d, pl.ds(seg * _SEG, _SEG)])
            _gather_rows(row_v, idx_v, out_v, write_seg)

        @pl.when(jnp.logical_not(is_user))
        def _():
            pltpu.sync_copy(igT_t.at[d], row_v)

            def write_seg(seg):
                pltpu.sync_copy(out_v, igT_o.at[d, pl.ds(seg * _SEG, _SEG)])
            _gather_rows(row_v, idx_v, out_v, write_seg)

    bpt = _B // (_NW // 2)
    base = local * bpt

    def beta_gather(beta_t, beta_o):
        copies = [
            pltpu.async_copy(beta_t.at[idx_v.at[pl.ds(base + c * 128, 128)]],
                             out_v.at[pl.ds(c * 128, 128)], sem)
            for c in range(bpt // 128)
        ]
        for c in copies:
            c.wait()
        pltpu.sync_copy(out_v.at[pl.ds(0, bpt)], beta_o.at[pl.ds(base, bpt)])

    @pl.when(is_user)
    def _():
        beta_gather(ub_t, ub_o)

    @pl.when(jnp.logical_not(is_user))
    def _():
        beta_gather(ib_t, ib_o)


_sc_gather = functools.partial(
    pl.kernel,
    mesh=plsc.VectorSubcoreMesh(core_axis_name="c", subcore_axis_name="s"),
    out_type=[
        jax.ShapeDtypeStruct((_D, _B), jnp.float32),
        jax.ShapeDtypeStruct((_D, _B), jnp.float32),
        jax.ShapeDtypeStruct((_B,), jnp.float32),
        jax.ShapeDtypeStruct((_B,), jnp.float32),
    ],
    scratch_types=[
        pltpu.VMEM((_V,), jnp.float32),
        pltpu.VMEM((_B,), jnp.int32),
        pltpu.VMEM((_SEG,), jnp.float32),
        pltpu.SemaphoreType.DMA,
    ],
    compiler_params=pltpu.CompilerParams(needs_layout_passes=False),
)(_gather_body)


def _math_body(ug_ref, ig_ref, ub_ref, ib_ref, out_ref):
    ug = ug_ref[...]
    ig = ig_ref[...]
    cu = jnp.maximum(jnp.sqrt(jnp.sum(ug * ug, axis=1, keepdims=True)), 1e-12)
    ci = jnp.maximum(jnp.sqrt(jnp.sum(ig * ig, axis=1, keepdims=True)), 1e-12)
    w = 1.0 / (cu * ci)
    wu = 1.0 / (cu * cu)
    wi = 1.0 / (ci * ci)
    num = jnp.sum(ug * ig * w, axis=0)
    rnu = jnp.sqrt(jnp.sum(ug * ug * wu, axis=0))
    rni = jnp.sqrt(jnp.sum(ig * ig * wi, axis=0))
    den = jnp.maximum(rnu, 1e-8) * jnp.maximum(rni, 1e-8)
    ub = ub_ref[...]
    ib = ib_ref[...]
    nbu = jnp.maximum(jnp.sqrt(jnp.sum(ub * ub)), 1e-12)
    nbi = jnp.maximum(jnp.sqrt(jnp.sum(ib * ib)), 1e-12)
    out_ref[...] = ib / nbi + ub / nbu + num / den


_tc_math = pl.pallas_call(
    _math_body,
    out_shape=jax.ShapeDtypeStruct((_B,), jnp.float32),
)


def kernel(users, items, user_gama, item_gama, user_beta, item_beta):
    users = users.astype(jnp.int32)
    items = items.astype(jnp.int32)
    ugT_t = user_gama.T
    igT_t = item_gama.T
    ub_t = user_beta.reshape(-1)
    ib_t = item_beta.reshape(-1)
    ugT, igT, ub, ib = _sc_gather(ugT_t, igT_t, ub_t, ib_t, users, items)
    return _tc_math(ugT, igT, ub, ib)
